# baseline (speedup 1.0000x reference)
import functools

import jax
import jax.numpy as jnp
from jax import lax
from jax.experimental import pallas as pl
from jax.experimental.pallas import tpu as pltpu
from jax.experimental.pallas import tpu_sc as plsc

_VOCAB = 1_000_000
_DIM = 32
_B = 4096
_CTX = 10
_NEG = 20

_info = plsc.get_sparse_core_info()
_NC = _info.num_cores
_NS = _info.num_subcores
_NW = _NC * _NS

_N_O = _B * _CTX
_N_N = _B * _CTX * _NEG
_B_PT = _B // _NW
_O_PT = _N_O // _NW
_N_PT = _N_N // _NW
_CHUNK = 2560
_NCH = _N_PT // _CHUNK


def _sc_gather_body(iword, owords, nwords, emb_in, emb_out,
                    iv_out, ov_out, nv_out,
                    idx_i, idx_o, idx_n, rows_v, sem):
    wid = lax.axis_index("s") * _NC + lax.axis_index("c")

    pltpu.sync_copy(iword.at[pl.ds(wid * _B_PT, _B_PT)], idx_i)
    pltpu.async_copy(emb_in.at[idx_i], rows_v.at[pl.ds(0, _B_PT)], sem).wait()
    pltpu.sync_copy(rows_v.at[pl.ds(0, _B_PT)],
                    iv_out.at[pl.ds(wid * _B_PT, _B_PT)])

    pltpu.sync_copy(owords.at[pl.ds(wid * _O_PT, _O_PT)], idx_o)
    pltpu.async_copy(emb_out.at[idx_o], rows_v.at[pl.ds(0, _O_PT)], sem).wait()
    pltpu.sync_copy(rows_v.at[pl.ds(0, _O_PT)],
                    ov_out.at[pl.ds(wid * _O_PT, _O_PT)])

    def body(cix, carry):
        base = wid * _N_PT + cix * _CHUNK
        pltpu.sync_copy(nwords.at[pl.ds(base, _CHUNK)], idx_n)
        pltpu.async_copy(emb_out.at[idx_n], rows_v, sem).wait()
        pltpu.sync_copy(rows_v, nv_out.at[pl.ds(base, _CHUNK)])
        return carry

    lax.fori_loop(0, _NCH, body, 0)


_sc_gather = functools.partial(
    pl.kernel,
    mesh=plsc.VectorSubcoreMesh(core_axis_name="c", subcore_axis_name="s"),
    compiler_params=pltpu.CompilerParams(use_tc_tiling_on_sc=False),
    out_type=[
        jax.ShapeDtypeStruct((_B, _DIM), jnp.float32),
        jax.ShapeDtypeStruct((_N_O, _DIM), jnp.float32),
        jax.ShapeDtypeStruct((_N_N, _DIM), jnp.float32),
    ],
    scratch_types=[
        pltpu.VMEM((_B_PT,), jnp.int32),
        pltpu.VMEM((_O_PT,), jnp.int32),
        pltpu.VMEM((_CHUNK,), jnp.int32),
        pltpu.VMEM((_CHUNK, _DIM), jnp.float32),
        pltpu.SemaphoreType.DMA,
    ],
)(_sc_gather_body)


_BB = 64


def _loss_body(iv_ref, ov_ref, nv_ref, out_ref):
    i = pl.program_id(0)
    iv = iv_ref[...]
    ov = ov_ref[...]
    nv = nv_ref[...]
    o_raw = jnp.sum(ov * iv[:, None, :], axis=-1)
    n_raw = jnp.sum(nv * iv[:, None, :], axis=-1)

    def softplus(x):
        return jnp.maximum(x, 0.0) + jnp.log(1.0 + jnp.exp(-jnp.abs(x)))

    s = jnp.sum(softplus(-o_raw)) + jnp.sum(softplus(n_raw))

    @pl.when(i == 0)
    def _():
        out_ref[0, 0] = 0.0

    out_ref[0, 0] += s


def _tc_loss(iv, ov3, nv3):
    out = pl.pallas_call(
        _loss_body,
        grid=(_B // _BB,),
        in_specs=[
            pl.BlockSpec((_BB, _DIM), lambda i: (i, 0)),
            pl.BlockSpec((_BB, _CTX, _DIM), lambda i: (i, 0, 0)),
            pl.BlockSpec((_BB, _CTX * _NEG, _DIM), lambda i: (i, 0, 0)),
        ],
        out_specs=pl.BlockSpec(memory_space=pltpu.SMEM),
        out_shape=jax.ShapeDtypeStruct((1, 1), jnp.float32),
    )(iv, ov3, nv3)
    return out[0, 0] / (_B * _CTX)


def kernel(iword, owords, nwords, emb_in, emb_out):
    iv, ov, nv = _sc_gather(iword, owords.reshape(-1), nwords.reshape(-1),
                            emb_in, emb_out)
    return _tc_loss(iv,
                    ov.reshape(_B, _CTX, _DIM),
                    nv.reshape(_B, _CTX * _NEG, _DIM))


# speedup vs baseline: 1.4541x; 1.4541x over previous
import functools

import jax
import jax.numpy as jnp
from jax import lax
from jax.experimental import pallas as pl
from jax.experimental.pallas import tpu as pltpu
from jax.experimental.pallas import tpu_sc as plsc

_VOCAB = 1_000_000
_DIM = 32
_B = 4096
_CTX = 10
_NEG = 20
_NNEG = _CTX * _NEG
_L = 16

_info = plsc.get_sparse_core_info()
_NC = _info.num_cores
_NS = _info.num_subcores
_NW = _NC * _NS

_B_PT = _B // _NW
_O_PT = _B_PT * _CTX
_N_PT = _B_PT * _NNEG

_G = 4
_NCHK = _B_PT // _G
_NROW = _G * _NNEG
_OROW = _G * _CTX

_NGRP = _NNEG // _L
_NPAD = (_NGRP + 1) * _L


def _sc_score_body(owords, nwords, iv, emb_out,
                   osc_out, nsc_out,
                   iv_rows, oidx, nidx, rows_o, rows_n,
                   osc_buf, nsc_buf, sem):
    wid = lax.axis_index("s") * _NC + lax.axis_index("c")
    b0 = wid * _B_PT
    pltpu.sync_copy(iv.at[pl.ds(b0, _B_PT)], iv_rows)
    pltpu.sync_copy(owords.at[pl.ds(wid * _O_PT, _O_PT)], oidx)
    lanes = lax.iota(jnp.int32, _L)

    def chunk(c, carry):
        pltpu.sync_copy(nwords.at[pl.ds(wid * _N_PT + c * _NROW, _NROW)],
                        nidx)
        cp_n = pltpu.async_copy(emb_out.at[nidx],
                                rows_n.at[pl.ds(0, _NROW)], sem)
        cp_o = pltpu.async_copy(emb_out.at[oidx.at[pl.ds(c * _OROW, _OROW)]],
                                rows_o.at[pl.ds(0, _OROW)], sem)
        cp_n.wait()
        cp_o.wait()

        def one_b(j, carry2):
            bl = c * _G + j
            for half in range(2):
                bc = [plsc.load_gather(
                          iv_rows,
                          [jnp.full((_L,), bl, jnp.int32),
                           jnp.full((_L,), half * _L + d, jnp.int32)])
                      for d in range(_L)]
                rb = j * _CTX
                acc = jnp.zeros((_L,), jnp.float32)
                for d in range(_L):
                    col = plsc.load_gather(
                        rows_o, [lanes + rb,
                                 jnp.full((_L,), half * _L + d, jnp.int32)])
                    acc = acc + col * bc[d]
                if half == 0:
                    osc_buf[bl, :] = acc
                else:
                    osc_buf[bl, :] += acc
                for g in range(_NGRP + 1):
                    rb = j * _NNEG + g * _L
                    acc = jnp.zeros((_L,), jnp.float32)
                    for d in range(_L):
                        col = plsc.load_gather(
                            rows_n, [lanes + rb,
                                     jnp.full((_L,), half * _L + d,
                                              jnp.int32)])
                        acc = acc + col * bc[d]
                    if half == 0:
                        nsc_buf[bl, pl.ds(g * _L, _L)] = acc
                    else:
                        nsc_buf[bl, pl.ds(g * _L, _L)] += acc
            return carry2

        lax.fori_loop(0, _G, one_b, 0)
        return carry

    lax.fori_loop(0, _NCHK, chunk, 0)
    pltpu.sync_copy(osc_buf, osc_out.at[pl.ds(b0, _B_PT)])
    pltpu.sync_copy(nsc_buf, nsc_out.at[pl.ds(b0, _B_PT)])


_sc_score = functools.partial(
    pl.kernel,
    mesh=plsc.VectorSubcoreMesh(core_axis_name="c", subcore_axis_name="s"),
    compiler_params=pltpu.CompilerParams(use_tc_tiling_on_sc=False,
                                         needs_layout_passes=False),
    out_type=[
        jax.ShapeDtypeStruct((_B, _L), jnp.float32),
        jax.ShapeDtypeStruct((_B, _NPAD), jnp.float32),
    ],
    scratch_types=[
        pltpu.VMEM((_B_PT, _DIM), jnp.float32),
        pltpu.VMEM((_O_PT,), jnp.int32),
        pltpu.VMEM((_NROW,), jnp.int32),
        pltpu.VMEM((_OROW + _L, _DIM), jnp.float32),
        pltpu.VMEM((_NROW + _L, _DIM), jnp.float32),
        pltpu.VMEM((_B_PT, _L), jnp.float32),
        pltpu.VMEM((_B_PT, _NPAD), jnp.float32),
        pltpu.SemaphoreType.DMA,
    ],
)(_sc_score_body)


def _loss_body(osc_ref, nsc_ref, out_ref):
    osc = osc_ref[...]
    nsc = nsc_ref[...]

    def softplus(x):
        return jnp.maximum(x, 0.0) + jnp.log(1.0 + jnp.exp(-jnp.abs(x)))

    ocol = lax.broadcasted_iota(jnp.int32, osc.shape, 1)
    ncol = lax.broadcasted_iota(jnp.int32, nsc.shape, 1)
    sp_o = jnp.where(ocol < _CTX, softplus(-osc), 0.0)
    sp_n = jnp.where(ncol < _NNEG, softplus(nsc), 0.0)
    out_ref[0, 0] = jnp.sum(sp_o) + jnp.sum(sp_n)


def _tc_loss(osc, nsc):
    out = pl.pallas_call(
        _loss_body,
        out_specs=pl.BlockSpec(memory_space=pltpu.SMEM),
        out_shape=jax.ShapeDtypeStruct((1, 1), jnp.float32),
    )(osc, nsc)
    return out[0, 0] / (_B * _CTX)


def kernel(iword, owords, nwords, emb_in, emb_out):
    iv = jnp.take(emb_in, iword, axis=0)
    osc, nsc = _sc_score(owords.reshape(-1), nwords.reshape(-1), iv, emb_out)
    return _tc_loss(osc, nsc)


# speedup vs baseline: 1.5551x; 1.0695x over previous
import functools

import jax
import jax.numpy as jnp
from jax import lax
from jax.experimental import pallas as pl
from jax.experimental.pallas import tpu as pltpu
from jax.experimental.pallas import tpu_sc as plsc

_VOCAB = 1_000_000
_DIM = 32
_B = 4096
_CTX = 10
_NEG = 20
_NNEG = _CTX * _NEG
_L = 16

_info = plsc.get_sparse_core_info()
_NC = _info.num_cores
_NS = _info.num_subcores
_NW = _NC * _NS

_B_PT = _B // _NW
_O_PT = _B_PT * _CTX
_N_PT = _B_PT * _NNEG

_G = 4
_NCHK = _B_PT // _G
_NROW = _G * _NNEG
_OROW = _G * _CTX

_NGRP = _NNEG // _L
_NPAD = (_NGRP + 1) * _L
_TS = _L + 1


def _sc_score_body(owords, nwords, iv, emb_out,
                   osc_out, nsc_out,
                   iv_rows, oidx, nidx, rows_o, rows_n, pscr,
                   osc_buf, nsc_buf, sem):
    wid = lax.axis_index("s") * _NC + lax.axis_index("c")
    b0 = wid * _B_PT
    pltpu.sync_copy(iv.at[pl.ds(b0, _B_PT)], iv_rows)
    pltpu.sync_copy(owords.at[pl.ds(wid * _O_PT, _O_PT)], oidx)
    lanes17 = lax.iota(jnp.int32, _L) * _TS

    def dot16(rows, rbase, ivlo, ivhi):
        for r in range(_L):
            lo = rows[rbase + r, pl.ds(0, _L)]
            hi = rows[rbase + r, pl.ds(_L, _L)]
            p = lo * ivlo + hi * ivhi
            plsc.store_scatter(pscr, [lanes17 + r], p)
        acc = pscr[pl.ds(0, _L)]
        for d in range(1, _L):
            acc = acc + pscr[pl.ds(d * _TS, _L)]
        return acc

    def chunk(c, carry):
        pltpu.sync_copy(nwords.at[pl.ds(wid * _N_PT + c * _NROW, _NROW)],
                        nidx)
        cp_n = pltpu.async_copy(emb_out.at[nidx],
                                rows_n.at[pl.ds(0, _NROW)], sem)
        cp_o = pltpu.async_copy(emb_out.at[oidx.at[pl.ds(c * _OROW, _OROW)]],
                                rows_o.at[pl.ds(0, _OROW)], sem)
        cp_n.wait()
        cp_o.wait()

        def one_b(j, carry2):
            bl = c * _G + j
            ivlo = iv_rows[bl, pl.ds(0, _L)]
            ivhi = iv_rows[bl, pl.ds(_L, _L)]
            osc_buf[bl, :] = dot16(rows_o, j * _CTX, ivlo, ivhi)
            for g in range(_NGRP + 1):
                nsc_buf[bl, pl.ds(g * _L, _L)] = dot16(
                    rows_n, j * _NNEG + g * _L, ivlo, ivhi)
            return carry2

        lax.fori_loop(0, _G, one_b, 0)
        return carry

    lax.fori_loop(0, _NCHK, chunk, 0)
    pltpu.sync_copy(osc_buf, osc_out.at[pl.ds(b0, _B_PT)])
    pltpu.sync_copy(nsc_buf, nsc_out.at[pl.ds(b0, _B_PT)])


_sc_score = functools.partial(
    pl.kernel,
    mesh=plsc.VectorSubcoreMesh(core_axis_name="c", subcore_axis_name="s"),
    compiler_params=pltpu.CompilerParams(use_tc_tiling_on_sc=False,
                                         needs_layout_passes=False),
    out_type=[
        jax.ShapeDtypeStruct((_B, _L), jnp.float32),
        jax.ShapeDtypeStruct((_B, _NPAD), jnp.float32),
    ],
    scratch_types=[
        pltpu.VMEM((_B_PT, _DIM), jnp.float32),
        pltpu.VMEM((_O_PT,), jnp.int32),
        pltpu.VMEM((_NROW,), jnp.int32),
        pltpu.VMEM((_OROW + _L, _DIM), jnp.float32),
        pltpu.VMEM((_NROW + _L, _DIM), jnp.float32),
        pltpu.VMEM((_L * _TS,), jnp.float32),
        pltpu.VMEM((_B_PT, _L), jnp.float32),
        pltpu.VMEM((_B_PT, _NPAD), jnp.float32),
        pltpu.SemaphoreType.DMA,
    ],
)(_sc_score_body)


def _loss_body(osc_ref, nsc_ref, out_ref):
    osc = osc_ref[...]
    nsc = nsc_ref[...]

    def softplus(x):
        return jnp.maximum(x, 0.0) + jnp.log(1.0 + jnp.exp(-jnp.abs(x)))

    ocol = lax.broadcasted_iota(jnp.int32, osc.shape, 1)
    ncol = lax.broadcasted_iota(jnp.int32, nsc.shape, 1)
    sp_o = jnp.where(ocol < _CTX, softplus(-osc), 0.0)
    sp_n = jnp.where(ncol < _NNEG, softplus(nsc), 0.0)
    out_ref[0, 0] = jnp.sum(sp_o) + jnp.sum(sp_n)


def _tc_loss(osc, nsc):
    out = pl.pallas_call(
        _loss_body,
        out_specs=pl.BlockSpec(memory_space=pltpu.SMEM),
        out_shape=jax.ShapeDtypeStruct((1, 1), jnp.float32),
    )(osc, nsc)
    return out[0, 0] / (_B * _CTX)


def kernel(iword, owords, nwords, emb_in, emb_out):
    iv = jnp.take(emb_in, iword, axis=0)
    osc, nsc = _sc_score(owords.reshape(-1), nwords.reshape(-1), iv, emb_out)
    return _tc_loss(osc, nsc)
